# baseline (speedup 1.0000x reference)
import jax
import jax.numpy as jnp
from jax import lax
from jax.experimental import pallas as pl
from jax.experimental.pallas import tpu as pltpu
from jax.experimental.pallas import tpu_sc as plsc

NUM_EMB = 1000000
DIM = 32
ROWS = 16384
COLS = 50
B_TOTAL = ROWS * COLS
NW = 32
RB = 16
CHUNK = RB * COLS
NBUF = 4
N_BLOCKS = ROWS // RB
BLK_PER_W = N_BLOCKS // NW
N_OUTER = BLK_PER_W // NBUF


def _emb_body(x_hbm, table_hbm, out_hbm, xstage, rows_v, isem, gsem, osem):
    wid = lax.axis_index("s") * 2 + lax.axis_index("c")
    base = wid * BLK_PER_W

    def start_idx_load(c, b):
        pltpu.async_copy(x_hbm.at[pl.ds((base + c) * RB, RB)], xstage.at[b],
                         isem.at[b])

    def wait_idx_load(c, b):
        pltpu.make_async_copy(x_hbm.at[pl.ds((base + c) * RB, RB)],
                              xstage.at[b], isem.at[b]).wait()

    def start_gathers(b):
        for k in range(RB):
            pltpu.async_copy(table_hbm.at[xstage.at[b, k]],
                             rows_v.at[b, pl.ds(k * COLS, COLS)], gsem.at[b])

    def wait_gathers(b):
        for k in range(RB):
            pltpu.make_async_copy(table_hbm.at[xstage.at[b, k]],
                                  rows_v.at[b, pl.ds(k * COLS, COLS)],
                                  gsem.at[b]).wait()

    def store_block(c, b):
        pltpu.async_copy(rows_v.at[b],
                         out_hbm.at[pl.ds((base + c) * CHUNK, CHUNK)],
                         osem.at[b])

    def wait_store(c, b):
        pltpu.make_async_copy(rows_v.at[b],
                              out_hbm.at[pl.ds((base + c) * CHUNK, CHUNK)],
                              osem.at[b]).wait()

    for b in range(NBUF):
        start_idx_load(b, b)
    for b in range(NBUF):
        wait_idx_load(b, b)
        start_gathers(b)

    def outer_step(o, carry):
        for b in range(NBUF):
            c = o * NBUF + b
            wait_gathers(b)
            store_block(c, b)

            @pl.when(c + NBUF < BLK_PER_W)
            def _():
                wait_store(c, b)
                start_idx_load(c + NBUF, b)
                wait_idx_load(c + NBUF, b)
                start_gathers(b)
        return carry

    lax.fori_loop(0, N_OUTER, outer_step, 0)

    for b in range(NBUF):
        wait_store(BLK_PER_W - NBUF + b, b)


@jax.jit
def _embedding_lookup(x, weight):
    mesh = plsc.VectorSubcoreMesh(core_axis_name="c", subcore_axis_name="s")
    run = pl.kernel(
        _emb_body,
        mesh=mesh,
        out_type=jax.ShapeDtypeStruct((B_TOTAL, DIM), jnp.float32),
        scratch_types=[
            pltpu.VMEM((NBUF, RB, COLS), jnp.int32),
            pltpu.VMEM((NBUF, CHUNK, DIM), jnp.float32),
            pltpu.SemaphoreType.DMA((NBUF,)),
            pltpu.SemaphoreType.DMA((NBUF,)),
            pltpu.SemaphoreType.DMA((NBUF,)),
        ],
        compiler_params=pltpu.CompilerParams(use_tc_tiling_on_sc=False),
    )
    return run(x, weight)


def kernel(x, weight):
    out = _embedding_lookup(x.astype(jnp.int32), weight)
    return out.reshape((ROWS, COLS, DIM))


# speedup vs baseline: 1.6228x; 1.6228x over previous
import jax
import jax.numpy as jnp
from jax import lax
from jax.experimental import pallas as pl
from jax.experimental.pallas import tpu as pltpu
from jax.experimental.pallas import tpu_sc as plsc

NUM_EMB = 1000000
DIM = 32
ROWS = 16384
COLS = 50
NW = 32
RB = 16
NBUF = 4
N_BLOCKS = ROWS // RB
BLK_PER_W = N_BLOCKS // NW


def _emb_body(x_hbm, table_hbm, out_hbm, xstage, rows_v, isem, gsem, osem):
    wid = lax.axis_index("s") * 2 + lax.axis_index("c")
    base = wid * BLK_PER_W

    def load_idx(c, b):
        pltpu.sync_copy(x_hbm.at[pl.ds((base + c) * RB, RB)], xstage.at[b])

    def start_gathers(b):
        def one(k, carry):
            pltpu.async_copy(table_hbm.at[xstage.at[b, k]],
                             rows_v.at[b, pl.ds(k * COLS, COLS)], gsem.at[b])
            return carry
        lax.fori_loop(0, RB, one, 0)

    def wait_gathers(b):
        def one(k, carry):
            pltpu.make_async_copy(table_hbm.at[xstage.at[b, k]],
                                  rows_v.at[b, pl.ds(k * COLS, COLS)],
                                  gsem.at[b]).wait()
            return carry
        lax.fori_loop(0, RB, one, 0)

    def start_stores(c, b):
        r0 = (base + c) * RB

        def one(k, carry):
            pltpu.async_copy(rows_v.at[b, pl.ds(k * COLS, COLS)],
                             out_hbm.at[r0 + k], osem.at[b])
            return carry
        lax.fori_loop(0, RB, one, 0)

    def wait_stores(c, b):
        r0 = (base + c) * RB

        def one(k, carry):
            pltpu.make_async_copy(rows_v.at[b, pl.ds(k * COLS, COLS)],
                                  out_hbm.at[r0 + k], osem.at[b]).wait()
            return carry
        lax.fori_loop(0, RB, one, 0)

    def prime(b, carry):
        load_idx(b, b)
        start_gathers(b)
        return carry

    lax.fori_loop(0, NBUF, prime, 0)

    def step(c, carry):
        b = lax.rem(c, NBUF)
        wait_gathers(b)
        start_stores(c, b)

        @pl.when(c + NBUF < BLK_PER_W)
        def _():
            wait_stores(c, b)
            load_idx(c + NBUF, b)
            start_gathers(b)
        return carry

    lax.fori_loop(0, BLK_PER_W, step, 0)

    def drain(b, carry):
        wait_stores(BLK_PER_W - NBUF + b, b)
        return carry

    lax.fori_loop(0, NBUF, drain, 0)


@jax.jit
def _embedding_lookup(x, weight):
    mesh = plsc.VectorSubcoreMesh(core_axis_name="c", subcore_axis_name="s")
    run = pl.kernel(
        _emb_body,
        mesh=mesh,
        out_type=jax.ShapeDtypeStruct((ROWS, COLS, DIM), jnp.float32),
        scratch_types=[
            pltpu.VMEM((NBUF, RB, COLS), jnp.int32),
            pltpu.VMEM((NBUF, RB * COLS, DIM), jnp.float32),
            pltpu.SemaphoreType.DMA((NBUF,)),
            pltpu.SemaphoreType.DMA((NBUF,)),
            pltpu.SemaphoreType.DMA((NBUF,)),
        ],
        compiler_params=pltpu.CompilerParams(use_tc_tiling_on_sc=False),
    )
    return run(x, weight)


def kernel(x, weight):
    return _embedding_lookup(x.astype(jnp.int32), weight)
